# Initial kernel scaffold; baseline (speedup 1.0000x reference)
import jax
import jax.numpy as jnp
from jax.experimental import pallas as pl


def kernel(x, edge_index, edge_type, node_graph_ids, head_ids, tail_ids, rel_labels, relation_triplets, rel_emb, W_rel, W_proj, b_proj, rel_vecs, W_self, W_nbr, W_fc, b_fc):
    raise NotImplementedError("write your pallas kernel here")


# speedup vs baseline: 5.3111x; 5.3111x over previous
import functools

import jax
import jax.numpy as jnp
from jax import lax
from jax.experimental import pallas as pl
from jax.experimental.pallas import tpu as pltpu
from jax.experimental.pallas import tpu_sc as plsc

N, E, D = 10000, 320000, 128
R, HR, RD = 32, 64, 32
B, L, T = 64, 3, 2048

NC, NS = 2, 16
NW = NC * NS
EW = E // NW
C = 80
NCH = EW // C
NPAD = 10240
TPT = NPAD // NS

_sc_mesh = plsc.VectorSubcoreMesh(core_axis_name="c", subcore_axis_name="s")
_sc_params = pltpu.CompilerParams(needs_layout_passes=False)

_f32 = jnp.float32


@functools.partial(
    pl.kernel,
    out_type=jax.ShapeDtypeStruct((NW, N), _f32),
    mesh=_sc_mesh,
    compiler_params=_sc_params,
    scratch_types=[
        pltpu.VMEM((EW,), jnp.int32),
        pltpu.VMEM((N,), _f32),
    ],
)
def _deg_sc(dst_hbm, out_hbm, dbuf, degt):
    c = lax.axis_index("c")
    s = lax.axis_index("s")
    wid = c * NS + s

    def zero(i, carry):
        degt[pl.ds(i * 16, 16)] = jnp.zeros((16,), _f32)
        return carry

    lax.fori_loop(0, N // 16, zero, 0)
    pltpu.sync_copy(dst_hbm.at[pl.ds(wid * EW, EW)], dbuf)
    ones16 = jnp.ones((16,), _f32)

    def scat(i, carry):
        idx = dbuf[pl.ds(i * 16, 16)]
        plsc.addupdate_scatter(degt, [idx], ones16)
        return carry

    lax.fori_loop(0, EW // 16, scat, 0)
    pltpu.sync_copy(degt, out_hbm.at[wid])


@functools.partial(
    pl.kernel,
    out_type=jax.ShapeDtypeStruct((NC * NPAD, D), _f32),
    mesh=_sc_mesh,
    compiler_params=_sc_params,
    scratch_types=[
        pltpu.VMEM((C,), jnp.int32),
        pltpu.VMEM((C,), jnp.int32),
        pltpu.VMEM((C,), jnp.int32),
        pltpu.VMEM((C, D), _f32),
        pltpu.VMEM((C, D), _f32),
        pltpu.VMEM_SHARED((R, D), _f32),
        pltpu.VMEM_SHARED((NPAD, D), _f32),
        pltpu.SemaphoreType.DMA,
        pltpu.SemaphoreType.DMA,
        pltpu.SemaphoreType.DMA,
        pltpu.SemaphoreType.DMA,
        pltpu.SemaphoreType.DMA,
    ],
)
def _edge_sc(h_hbm, src_hbm, dst_hbm, et_hbm, rv_hbm, out_hbm,
             src80, et80, dst80, rows, rvrows,
             rv_sh, agg_sh, gsem1, gsem2, isem1, isem2, isem3):
    c = lax.axis_index("c")
    s = lax.axis_index("s")
    wid = c * NS + s
    ebase = wid * EW

    def zrow(i, carry):
        rows[i // 8, pl.ds((i % 8) * 16, 16)] = jnp.zeros((16,), _f32)
        return carry

    lax.fori_loop(0, C * 8, zrow, 0)

    def zcp(k, carry):
        pltpu.sync_copy(rows, agg_sh.at[pl.ds(s * TPT + k * C, C)])
        return carry

    lax.fori_loop(0, TPT // C, zcp, 0)

    @pl.when(s == 0)
    def _():
        pltpu.sync_copy(rv_hbm, rv_sh)

    plsc.subcore_barrier()

    def chunk(i, carry):
        off = ebase + i * C
        i1 = pltpu.async_copy(src_hbm.at[pl.ds(off, C)], src80, isem1)
        i2 = pltpu.async_copy(dst_hbm.at[pl.ds(off, C)], dst80, isem2)
        i3 = pltpu.async_copy(et_hbm.at[pl.ds(off, C)], et80, isem3)
        i1.wait()
        i3.wait()
        g1 = pltpu.async_copy(h_hbm.at[src80], rows, gsem1)
        g2 = pltpu.async_copy(rv_sh.at[et80], rvrows, gsem2)
        g1.wait()
        g2.wait()

        def mul(j, carry2):
            for k in range(D // 16):
                sl = pl.ds(k * 16, 16)
                rows[j, sl] = rows[j, sl] * rvrows[j, sl]
            return carry2

        lax.fori_loop(0, C, mul, 0)
        i2.wait()
        pltpu.sync_copy(rows, agg_sh.at[dst80], add=True)
        return carry

    lax.fori_loop(0, NCH, chunk, 0)
    plsc.subcore_barrier()
    pltpu.sync_copy(agg_sh.at[pl.ds(s * TPT, TPT)],
                    out_hbm.at[pl.ds(c * NPAD + s * TPT, TPT)])


def _rel_body(srcr_ref, dstr_ref, rel_emb_ref, wrel_ref, wproj_ref, bproj_ref,
              out_ref):
    srcr = srcr_ref[...]
    dstr = dstr_ref[...]
    iota_r = lax.broadcasted_iota(jnp.int32, (R, T), 0)
    ohs = (iota_r == srcr).astype(_f32)
    ohd = (iota_r == dstr).astype(_f32)
    p = lax.dot_general(ohd, ohs, (((1,), (1,)), ((), ())),
                        preferred_element_type=_f32)
    cnt = jnp.sum(p, axis=1, keepdims=True)
    agg = jnp.dot(p, rel_emb_ref[...], preferred_element_type=_f32)
    agg = agg / jnp.maximum(cnt, 1.0)
    emb = jnp.maximum(
        jnp.dot(rel_emb_ref[...] + agg, wrel_ref[...],
                preferred_element_type=_f32), 0.0)
    out_ref[...] = jnp.maximum(
        jnp.dot(emb, wproj_ref[...], preferred_element_type=_f32)
        + bproj_ref[...], 0.0)


_rel_tc = pl.pallas_call(
    _rel_body, out_shape=jax.ShapeDtypeStruct((R, RD), _f32))


def _layer_body(h_ref, a_ref, degp_ref, ws_ref, wn_ref, o_ref):
    deg = jnp.sum(degp_ref[...], axis=1, keepdims=True)
    norm = 1.0 / jnp.maximum(deg, 1.0)
    a = a_ref[...]
    agg = (a[0:N] + a[NPAD:NPAD + N]) * norm
    o_ref[...] = jnp.maximum(
        jnp.dot(h_ref[...], ws_ref[...], preferred_element_type=_f32)
        + jnp.dot(agg, wn_ref[...], preferred_element_type=_f32), 0.0)


_layer_tc = pl.pallas_call(
    _layer_body, out_shape=jax.ShapeDtypeStruct((N, D), _f32))


def _final_body(h1_ref, h2_ref, h3_ref, gid_ref, hid_ref, tid_ref, rl_ref,
                emb_rel_ref, wfc_ref, bfc_ref, out_ref):
    gid = gid_ref[...]
    ohg = (lax.broadcasted_iota(jnp.int32, (B, N), 0) == gid).astype(_f32)
    gcnt = jnp.sum(ohg, axis=1, keepdims=True)
    ginv = 1.0 / jnp.maximum(gcnt, 1.0)
    iota_n = lax.broadcasted_iota(jnp.int32, (B, N), 1)
    ohh = (iota_n == hid_ref[...]).astype(_f32)
    oht = (iota_n == tid_ref[...]).astype(_f32)
    ohr = (lax.broadcasted_iota(jnp.int32, (B, R), 1)
           == rl_ref[...]).astype(_f32)
    wfc = wfc_ref[...]
    bfc = bfc_ref[...]

    hs = (h1_ref[...], h2_ref[...], h3_ref[...])
    acc = jnp.zeros((B, 1), _f32)
    for j in range(L):
        hj = hs[j]
        gj = jnp.dot(ohg, hj, preferred_element_type=_f32) * ginv
        acc = acc + jnp.dot(gj, wfc[j * D:(j + 1) * D],
                            preferred_element_type=_f32)
        hd = jnp.dot(ohh, hj, preferred_element_type=_f32)
        acc = acc + jnp.dot(hd, wfc[L * D + j * D:L * D + (j + 1) * D],
                            preferred_element_type=_f32)
        tl = jnp.dot(oht, hj, preferred_element_type=_f32)
        acc = acc + jnp.dot(tl, wfc[2 * L * D + j * D:2 * L * D + (j + 1) * D],
                            preferred_element_type=_f32)
    emb_sel = jnp.dot(ohr, emb_rel_ref[...], preferred_element_type=_f32)
    acc = acc + jnp.dot(emb_sel, wfc[3 * L * D:3 * L * D + RD],
                        preferred_element_type=_f32)
    out_ref[...] = acc + bfc


_final_tc = pl.pallas_call(
    _final_body, out_shape=jax.ShapeDtypeStruct((B, 1), _f32))


def kernel(x, edge_index, edge_type, node_graph_ids, head_ids, tail_ids,
           rel_labels, relation_triplets, rel_emb, W_rel, W_proj, b_proj,
           rel_vecs, W_self, W_nbr, W_fc, b_fc):
    src = edge_index[0]
    dst = edge_index[1]
    degp = _deg_sc(dst)
    degp_t = degp.T

    emb_rel = _rel_tc(relation_triplets[:, 0].reshape(1, T).astype(jnp.int32),
                      relation_triplets[:, 2].reshape(1, T).astype(jnp.int32),
                      rel_emb, W_rel, W_proj, b_proj.reshape(1, RD))

    h = x
    hs = []
    for l in range(L):
        aggp = _edge_sc(h, src, dst, edge_type, rel_vecs[l])
        h = _layer_tc(h, aggp, degp_t, W_self[l], W_nbr[l])
        hs.append(h)

    out = _final_tc(hs[0], hs[1], hs[2],
                    node_graph_ids.reshape(1, N).astype(jnp.int32),
                    head_ids.reshape(B, 1).astype(jnp.int32),
                    tail_ids.reshape(B, 1).astype(jnp.int32),
                    rel_labels.reshape(B, 1).astype(jnp.int32),
                    emb_rel, W_fc, b_fc.reshape(1, 1))
    return out


# speedup vs baseline: 8.0227x; 1.5106x over previous
import functools

import jax
import jax.numpy as jnp
from jax import lax
from jax.experimental import pallas as pl
from jax.experimental.pallas import tpu as pltpu
from jax.experimental.pallas import tpu_sc as plsc

N, E, D = 10000, 320000, 128
R, HR, RD = 32, 64, 32
B, L, T = 64, 3, 2048

NC, NS = 2, 16
NW = NC * NS
EW = E // NW
C = 80
NCH = EW // C
NPAD = 10240
TPT = NPAD // NS

_sc_mesh = plsc.VectorSubcoreMesh(core_axis_name="c", subcore_axis_name="s")
_sc_params = pltpu.CompilerParams(needs_layout_passes=False)

_f32 = jnp.float32


@functools.partial(
    pl.kernel,
    out_type=jax.ShapeDtypeStruct((NW, N), _f32),
    mesh=_sc_mesh,
    compiler_params=_sc_params,
    scratch_types=[
        pltpu.VMEM((EW,), jnp.int32),
        pltpu.VMEM((N,), _f32),
    ],
)
def _deg_sc(dst_hbm, out_hbm, dbuf, degt):
    c = lax.axis_index("c")
    s = lax.axis_index("s")
    wid = c * NS + s

    def zero(i, carry):
        degt[pl.ds(i * 16, 16)] = jnp.zeros((16,), _f32)
        return carry

    lax.fori_loop(0, N // 16, zero, 0)
    pltpu.sync_copy(dst_hbm.at[pl.ds(wid * EW, EW)], dbuf)
    ones16 = jnp.ones((16,), _f32)

    def scat(i, carry):
        idx = dbuf[pl.ds(i * 16, 16)]
        plsc.addupdate_scatter(degt, [idx], ones16)
        return carry

    lax.fori_loop(0, EW // 16, scat, 0)
    pltpu.sync_copy(degt, out_hbm.at[wid])


@functools.partial(
    pl.kernel,
    out_type=jax.ShapeDtypeStruct((NC * NPAD, D), _f32),
    mesh=_sc_mesh,
    compiler_params=_sc_params,
    scratch_types=[
        pltpu.VMEM((C,), jnp.int32),
        pltpu.VMEM((C,), jnp.int32),
        pltpu.VMEM((C,), jnp.int32),
        pltpu.VMEM((C,), jnp.int32),
        pltpu.VMEM((C,), jnp.int32),
        pltpu.VMEM((C,), jnp.int32),
        pltpu.VMEM((C, D), _f32),
        pltpu.VMEM((C, D), _f32),
        pltpu.VMEM((C, D), _f32),
        pltpu.VMEM((C, D), _f32),
        pltpu.VMEM_SHARED((R, D), _f32),
        pltpu.VMEM_SHARED((NPAD, D), _f32),
        pltpu.SemaphoreType.DMA,
        pltpu.SemaphoreType.DMA,
        pltpu.SemaphoreType.DMA,
        pltpu.SemaphoreType.DMA,
        pltpu.SemaphoreType.DMA,
        pltpu.SemaphoreType.DMA,
        pltpu.SemaphoreType.DMA,
        pltpu.SemaphoreType.DMA,
        pltpu.SemaphoreType.DMA,
        pltpu.SemaphoreType.DMA,
    ],
)
def _edge_sc(h_hbm, src_hbm, dst_hbm, et_hbm, rv_hbm, out_hbm,
             srcA, etA, dstA, srcB, etB, dstB,
             rowsA, rvA, rowsB, rvB, rv_sh, agg_sh,
             ssemA, esemA, dsemA, ssemB, esemB, dsemB,
             gsemA, vsemA, gsemB, vsemB):
    c = lax.axis_index("c")
    s = lax.axis_index("s")
    wid = c * NS + s
    ebase = wid * EW

    def zrow(i, carry):
        rowsA[i // 8, pl.ds((i % 8) * 16, 16)] = jnp.zeros((16,), _f32)
        return carry

    lax.fori_loop(0, C * 8, zrow, 0)

    def zcp(k, carry):
        pltpu.sync_copy(rowsA, agg_sh.at[pl.ds(s * TPT + k * C, C)])
        return carry

    lax.fori_loop(0, TPT // C, zcp, 0)

    @pl.when(s == 0)
    def _():
        pltpu.sync_copy(rv_hbm, rv_sh)

    plsc.subcore_barrier()

    bufs = (
        (srcA, etA, dstA, rowsA, rvA, ssemA, esemA, dsemA, gsemA, vsemA),
        (srcB, etB, dstB, rowsB, rvB, ssemB, esemB, dsemB, gsemB, vsemB),
    )

    def idx_start(i, b):
        src, et, dst, _, _, ssem, esem, dsem, _, _ = bufs[b]
        off = ebase + i * C
        pltpu.async_copy(src_hbm.at[pl.ds(off, C)], src, ssem)
        pltpu.async_copy(et_hbm.at[pl.ds(off, C)], et, esem)
        pltpu.async_copy(dst_hbm.at[pl.ds(off, C)], dst, dsem)

    def gather_start(b):
        src, et, _, rows, rv, ssem, esem, _, gsem, vsem = bufs[b]
        pltpu.make_async_copy(src_hbm.at[pl.ds(ebase, C)], src, ssem).wait()
        pltpu.make_async_copy(et_hbm.at[pl.ds(ebase, C)], et, esem).wait()
        pltpu.async_copy(h_hbm.at[src], rows, gsem)
        pltpu.async_copy(rv_sh.at[et], rv, vsem)

    def mul_scatter(b):
        _, _, dst, rows, rv, _, _, dsem, gsem, vsem = bufs[b]
        pltpu.make_async_copy(h_hbm.at[pl.ds(0, C)], rows, gsem).wait()
        pltpu.make_async_copy(h_hbm.at[pl.ds(0, C)], rv, vsem).wait()

        @plsc.parallel_loop(0, C, 1, unroll=2)
        def mul(j):
            for k in range(D // 16):
                sl = pl.ds(k * 16, 16)
                rows[j, sl] = rows[j, sl] * rv[j, sl]

        pltpu.make_async_copy(dst_hbm.at[pl.ds(ebase, C)], dst, dsem).wait()
        pltpu.sync_copy(rows, agg_sh.at[dst], add=True)

    idx_start(0, 0)
    idx_start(1, 1)
    gather_start(0)

    def pair(t, carry):
        i = t * 2
        gather_start(1)
        mul_scatter(0)
        idx_start(i + 2, 0)
        gather_start(0)
        mul_scatter(1)

        @pl.when(i + 3 <= NCH - 1)
        def _():
            idx_start(i + 3, 1)

        return carry

    lax.fori_loop(0, (NCH - 1) // 2, pair, 0)
    mul_scatter(0)
    plsc.subcore_barrier()
    pltpu.sync_copy(agg_sh.at[pl.ds(s * TPT, TPT)],
                    out_hbm.at[pl.ds(c * NPAD + s * TPT, TPT)])


def _rel_body(srcr_ref, dstr_ref, rel_emb_ref, wrel_ref, wproj_ref, bproj_ref,
              out_ref):
    srcr = srcr_ref[...]
    dstr = dstr_ref[...]
    iota_r = lax.broadcasted_iota(jnp.int32, (R, T), 0)
    ohs = (iota_r == srcr).astype(_f32)
    ohd = (iota_r == dstr).astype(_f32)
    p = lax.dot_general(ohd, ohs, (((1,), (1,)), ((), ())),
                        preferred_element_type=_f32)
    cnt = jnp.sum(p, axis=1, keepdims=True)
    agg = jnp.dot(p, rel_emb_ref[...], preferred_element_type=_f32)
    agg = agg / jnp.maximum(cnt, 1.0)
    emb = jnp.maximum(
        jnp.dot(rel_emb_ref[...] + agg, wrel_ref[...],
                preferred_element_type=_f32), 0.0)
    out_ref[...] = jnp.maximum(
        jnp.dot(emb, wproj_ref[...], preferred_element_type=_f32)
        + bproj_ref[...], 0.0)


_rel_tc = pl.pallas_call(
    _rel_body, out_shape=jax.ShapeDtypeStruct((R, RD), _f32))


def _layer_body(h_ref, a_ref, degp_ref, ws_ref, wn_ref, o_ref):
    deg = jnp.sum(degp_ref[...], axis=1, keepdims=True)
    norm = 1.0 / jnp.maximum(deg, 1.0)
    a = a_ref[...]
    agg = (a[0:N] + a[NPAD:NPAD + N]) * norm
    o_ref[...] = jnp.maximum(
        jnp.dot(h_ref[...], ws_ref[...], preferred_element_type=_f32)
        + jnp.dot(agg, wn_ref[...], preferred_element_type=_f32), 0.0)


_layer_tc = pl.pallas_call(
    _layer_body, out_shape=jax.ShapeDtypeStruct((N, D), _f32))


def _final_body(h1_ref, h2_ref, h3_ref, gid_ref, hid_ref, tid_ref, rl_ref,
                emb_rel_ref, wfc_ref, bfc_ref, out_ref):
    gid = gid_ref[...]
    ohg = (lax.broadcasted_iota(jnp.int32, (B, N), 0) == gid).astype(_f32)
    gcnt = jnp.sum(ohg, axis=1, keepdims=True)
    ginv = 1.0 / jnp.maximum(gcnt, 1.0)
    iota_n = lax.broadcasted_iota(jnp.int32, (B, N), 1)
    ohh = (iota_n == hid_ref[...]).astype(_f32)
    oht = (iota_n == tid_ref[...]).astype(_f32)
    ohr = (lax.broadcasted_iota(jnp.int32, (B, R), 1)
           == rl_ref[...]).astype(_f32)
    wfc = wfc_ref[...]
    bfc = bfc_ref[...]

    hs = (h1_ref[...], h2_ref[...], h3_ref[...])
    acc = jnp.zeros((B, 1), _f32)
    for j in range(L):
        hj = hs[j]
        gj = jnp.dot(ohg, hj, preferred_element_type=_f32) * ginv
        acc = acc + jnp.dot(gj, wfc[j * D:(j + 1) * D],
                            preferred_element_type=_f32)
        hd = jnp.dot(ohh, hj, preferred_element_type=_f32)
        acc = acc + jnp.dot(hd, wfc[L * D + j * D:L * D + (j + 1) * D],
                            preferred_element_type=_f32)
        tl = jnp.dot(oht, hj, preferred_element_type=_f32)
        acc = acc + jnp.dot(tl, wfc[2 * L * D + j * D:2 * L * D + (j + 1) * D],
                            preferred_element_type=_f32)
    emb_sel = jnp.dot(ohr, emb_rel_ref[...], preferred_element_type=_f32)
    acc = acc + jnp.dot(emb_sel, wfc[3 * L * D:3 * L * D + RD],
                        preferred_element_type=_f32)
    out_ref[...] = acc + bfc


_final_tc = pl.pallas_call(
    _final_body, out_shape=jax.ShapeDtypeStruct((B, 1), _f32))


def kernel(x, edge_index, edge_type, node_graph_ids, head_ids, tail_ids,
           rel_labels, relation_triplets, rel_emb, W_rel, W_proj, b_proj,
           rel_vecs, W_self, W_nbr, W_fc, b_fc):
    src = edge_index[0]
    dst = edge_index[1]
    degp = _deg_sc(dst)
    degp_t = degp.T

    emb_rel = _rel_tc(relation_triplets[:, 0].reshape(1, T).astype(jnp.int32),
                      relation_triplets[:, 2].reshape(1, T).astype(jnp.int32),
                      rel_emb, W_rel, W_proj, b_proj.reshape(1, RD))

    h = x
    hs = []
    for l in range(L):
        aggp = _edge_sc(h, src, dst, edge_type, rel_vecs[l])
        h = _layer_tc(h, aggp, degp_t, W_self[l], W_nbr[l])
        hs.append(h)

    out = _final_tc(hs[0], hs[1], hs[2],
                    node_graph_ids.reshape(1, N).astype(jnp.int32),
                    head_ids.reshape(B, 1).astype(jnp.int32),
                    tail_ids.reshape(B, 1).astype(jnp.int32),
                    rel_labels.reshape(B, 1).astype(jnp.int32),
                    emb_rel, W_fc, b_fc.reshape(1, 1))
    return out
